# baseline (speedup 1.0000x reference)
import functools

import numpy as np
import jax
import jax.numpy as jnp
from jax.experimental import pallas as pl
from jax.experimental.pallas import tpu as pltpu

_RADIUS = 0.5
_H = 4
_C = 32
_G = 16


def _sel_pack(n, h, c):
    rows = np.arange(n * h * c)
    d, hh = rows // (h * c), (rows % (h * c)) // c
    m = np.zeros((n * h * c, h * n), np.float32)
    m[rows, hh * n + d] = 1.0
    return m


def _sel_spread(n, h, c):
    m = np.zeros((h * n, n * h * c), np.float32)
    for hh in range(h):
        for d in range(n):
            m[hh * n + d, d * h * c + hh * c:d * h * c + (hh + 1) * c] = 1.0
    return m


def _gnn_kernel(x_ref, a_ref, w1_ref, b1_ref, w2_ref, b2_ref, wq_ref,
                bq_ref, wk_ref, bk_ref, wv_ref, bv_ref, we_ref, wskip_ref,
                bskip_ref, hs_ref, hst_ref, sumg_ref, rspread_ref, out_ref,
                w2e8_s, wcat_s, bcat_s, xemb_s):
    G, N, D = x_ref.shape
    EH = 32
    HC = we_ref.shape[1]
    NE = N * EH
    BD = 8
    f32 = jnp.float32
    hi = jax.lax.Precision.DEFAULT

    @pl.when(pl.program_id(0) == 0)
    def _setup():
        w2rep = jnp.tile(w2_ref[...], (BD, BD))
        rr = jax.lax.broadcasted_iota(jnp.int32, (BD * EH, BD * EH), 0) // EH
        cc = jax.lax.broadcasted_iota(jnp.int32, (BD * EH, BD * EH), 1) // EH
        w2e8_s[...] = w2rep * (rr == cc).astype(f32)
        wcat_s[...] = jnp.concatenate(
            [wq_ref[...], wk_ref[...], wv_ref[...], wskip_ref[...]], axis=1)
        bcat_s[0:1, :] = jnp.concatenate(
            [bq_ref[...], bk_ref[...], bv_ref[...], bskip_ref[...]], axis=1)

    a2 = a_ref[...].reshape(G * N, N)
    maskf2 = ((a2 > 0.0) & (a2 < _RADIUS)).astype(f32)
    am2 = a2 * maskf2

    x2 = x_ref[...].reshape(G * N, D)
    t1n = jnp.dot(x2, w1_ref[0:D, :], precision=hi) + b1_ref[...]
    amt = jnp.dot(am2, rspread_ref[...], precision=hi)
    h1w = jnp.maximum(jnp.tile(t1n, (1, N))
                      + amt * jnp.tile(w1_ref[D:D + 1, :], (1, N)), 0.0)
    b2rep = jnp.tile(b2_ref[...], (1, BD))
    h2w = jnp.concatenate([
        jnp.maximum(jnp.dot(h1w[:, c * BD * EH:(c + 1) * BD * EH],
                            w2e8_s[...], precision=hi) + b2rep, 0.0)
        for c in range(N // BD)], axis=1)
    h2w = h2w * (amt > 0.0).astype(f32)
    x_embw = jnp.dot(sumg_ref[...], h2w, precision=hi)
    xemb_s[...] = x_embw.reshape(G, N, EH)
    x_emb = xemb_s[...].reshape(G * N, EH)

    proj = jnp.dot(x_emb, wcat_s[...], precision=hi) + bcat_s[0:1, :]
    q = proj[:, 0 * HC:1 * HC]
    k = proj[:, 1 * HC:2 * HC].reshape(G, N, HC)
    v = proj[:, 2 * HC:3 * HC].reshape(G, N, HC)
    skip = proj[:, 3 * HC:4 * HC]

    we = we_ref[...]

    maskfp = jnp.tile(maskf2, (1, _H)).reshape(G, N, HC)
    amp = jnp.tile(am2, (1, _H)).reshape(G, N, HC)

    qwe = jnp.dot(q * we, hs_ref[...], precision=hi)
    qwe_p = jnp.swapaxes(qwe.reshape(G, N, _H), 1, 2).reshape(G, _H * N)
    qT = jnp.swapaxes(q.reshape(G, N, HC), 1, 2)
    logits_p = jnp.stack([
        jnp.concatenate([
            jnp.dot(k[g, :, h * _C:(h + 1) * _C],
                    qT[g, h * _C:(h + 1) * _C, :], precision=hi)
            for h in range(_H)], axis=1)
        for g in range(G)], axis=0)
    scale = 1.0 / np.sqrt(float(_C))
    logits = (logits_p + amp * qwe_p[:, None, :]) * scale

    lm = logits + (maskfp - 1.0) * 1e30
    amax = jnp.max(lm, axis=1, keepdims=True)
    amax = jnp.where(amax > -1e29, amax, 0.0)
    ex = jnp.exp(logits - amax) * maskfp
    den = jnp.sum(ex, axis=1, keepdims=True)
    den = jnp.where(den > 0.0, den, 1.0)
    alpha_p = ex / den

    bw = jnp.sum(alpha_p * amp, axis=1)
    bw4 = jnp.swapaxes(bw.reshape(G, _H, N), 1, 2).reshape(G * N, _H)
    bw128 = jnp.dot(bw4, hst_ref[...], precision=hi)

    alphaT = jnp.swapaxes(alpha_p, 1, 2)
    out_v = jnp.stack([
        jnp.concatenate([
            jnp.dot(alphaT[g, h * N:(h + 1) * N, :],
                    v[g, :, h * _C:(h + 1) * _C], precision=hi)
            for h in range(_H)], axis=1)
        for g in range(G)], axis=0)
    out_v = out_v.reshape(G * N, HC)

    out_ref[...] = jnp.maximum(out_v + bw128 * we + skip, 0.0)


@functools.partial(jax.jit, static_argnames=())
def kernel(node_obs, adj, W1, b1, W2, b2, Wq, bq, Wk, bk, Wv, bv, We,
           Wskip, bskip):
    M, N, D = node_obs.shape
    EH = W2.shape[0]
    HC = We.shape[1]
    G = _G
    f32 = jnp.float32

    hs = jnp.asarray(np.kron(np.eye(_H), np.ones((_C, 1))), f32)
    hst = jnp.asarray(np.kron(np.eye(_H), np.ones((1, _C))), f32)
    sumg = jnp.asarray(np.kron(np.eye(G), np.ones((1, N))), f32)
    rspread = jnp.asarray(np.kron(np.eye(N), np.ones((1, EH))), f32)

    full = lambda shape: pl.BlockSpec(shape, lambda i: (0,) * len(shape))
    out = pl.pallas_call(
        _gnn_kernel,
        grid=(M // G,),
        in_specs=[
            pl.BlockSpec((G, N, D), lambda i: (i, 0, 0)),
            pl.BlockSpec((G, N, N), lambda i: (i, 0, 0)),
            full((D + 1, EH)),
            full((1, EH)),
            full((EH, EH)),
            full((1, EH)),
            full((EH, HC)),
            full((1, HC)),
            full((EH, HC)),
            full((1, HC)),
            full((EH, HC)),
            full((1, HC)),
            full((1, HC)),
            full((EH, HC)),
            full((1, HC)),
            full((HC, _H)),
            full((_H, HC)),
            full((G, G * N)),
            full((N, N * EH)),
        ],
        out_specs=pl.BlockSpec((G * N, HC), lambda i: (i, 0)),
        out_shape=jax.ShapeDtypeStruct((M * N, HC), jnp.float32),
        scratch_shapes=[
            pltpu.VMEM((8 * EH, 8 * EH), f32),
            pltpu.VMEM((EH, 4 * HC), f32),
            pltpu.VMEM((8, 4 * HC), f32),
            pltpu.VMEM((G, N, EH), f32),
        ],
    )(node_obs, adj, W1, b1.reshape(1, EH), W2, b2.reshape(1, EH),
      Wq, bq.reshape(1, HC), Wk, bk.reshape(1, HC), Wv, bv.reshape(1, HC),
      We, Wskip, bskip.reshape(1, HC), hs, hst, sumg, rspread)
    return out


# speedup vs baseline: 1.0784x; 1.0784x over previous
import functools

import numpy as np
import jax
import jax.numpy as jnp
from jax.experimental import pallas as pl
from jax.experimental.pallas import tpu as pltpu

_RADIUS = 0.5
_H = 4
_C = 32
_G = 16


def _gnn_kernel(x_ref, a_ref, w1_ref, b1_ref, w2_ref, b2_ref, wq_ref,
                bq_ref, wk_ref, bk_ref, wv_ref, bv_ref, we_ref, wskip_ref,
                bskip_ref, hs_ref, hst_ref, out_ref,
                w1big_s, misc_s, w2i_s, wcat_s, bcat_s):
    G, N, D = x_ref.shape
    EH = 32
    HC = we_ref.shape[1]
    NE = N * EH
    f32 = jnp.float32
    hi = jax.lax.Precision.DEFAULT

    @pl.when(pl.program_id(0) == 0)
    def _setup():
        w1big_s[...] = jnp.repeat(w1_ref[0:D, :], N, axis=1)
        misc_s[0:1, :] = jnp.repeat(w1_ref[D:D + 1, :], N, axis=1)
        misc_s[1:2, :] = jnp.repeat(b1_ref[...], N, axis=1)
        misc_s[2:3, :] = jnp.repeat(b2_ref[...], N, axis=1)
        w2rep = jnp.repeat(jnp.repeat(w2_ref[...], N, axis=0), N, axis=1)
        rr = jax.lax.broadcasted_iota(jnp.int32, (NE, NE), 0) % N
        cc = jax.lax.broadcasted_iota(jnp.int32, (NE, NE), 1) % N
        w2i_s[...] = w2rep * (rr == cc).astype(f32)
        wcat_s[...] = jnp.concatenate(
            [wq_ref[...], wk_ref[...], wv_ref[...], wskip_ref[...]], axis=1)
        bcat_s[0:1, :] = jnp.concatenate(
            [bq_ref[...], bk_ref[...], bv_ref[...], bskip_ref[...]], axis=1)

    a2 = a_ref[...].reshape(G * N, N)
    maskf2 = ((a2 > 0.0) & (a2 < _RADIUS)).astype(f32)
    am2 = a2 * maskf2

    x2 = x_ref[...].reshape(G * N, D)
    t1w = jnp.dot(x2, w1big_s[...], precision=hi) + misc_s[1:2, :]
    amt = jnp.tile(am2, (1, EH))
    h1w = jnp.maximum(t1w + amt * misc_s[0:1, :], 0.0)
    h2w = jnp.maximum(jnp.dot(h1w, w2i_s[...], precision=hi)
                      + misc_s[2:3, :], 0.0)
    h2w = h2w * jnp.tile(maskf2, (1, EH))
    x_embw = jnp.sum(h2w.reshape(G, N, NE), axis=1)
    x_emb = jnp.swapaxes(x_embw.reshape(G, EH, N), 1, 2).reshape(G * N, EH)

    proj = jnp.dot(x_emb, wcat_s[...], precision=hi) + bcat_s[0:1, :]
    q = proj[:, 0 * HC:1 * HC]
    k = proj[:, 1 * HC:2 * HC].reshape(G, N, HC)
    v = proj[:, 2 * HC:3 * HC].reshape(G, N, HC)
    skip = proj[:, 3 * HC:4 * HC]

    we = we_ref[...]

    maskfp = jnp.tile(maskf2, (1, _H)).reshape(G, N, HC)
    amp = jnp.tile(am2, (1, _H)).reshape(G, N, HC)

    qwe = jnp.dot(q * we, hs_ref[...], precision=hi)
    qwe_p = jnp.swapaxes(qwe.reshape(G, N, _H), 1, 2).reshape(G, _H * N)
    qT = jnp.swapaxes(q.reshape(G, N, HC), 1, 2)
    logits_p = jnp.stack([
        jnp.concatenate([
            jnp.dot(k[g, :, h * _C:(h + 1) * _C],
                    qT[g, h * _C:(h + 1) * _C, :], precision=hi)
            for h in range(_H)], axis=1)
        for g in range(G)], axis=0)
    scale = 1.0 / np.sqrt(float(_C))
    logits = (logits_p + amp * qwe_p[:, None, :]) * scale

    lm = logits + (maskfp - 1.0) * 1e30
    amax = jnp.max(lm, axis=1, keepdims=True)
    amax = jnp.where(amax > -1e29, amax, 0.0)
    ex = jnp.exp(logits - amax) * maskfp
    den = jnp.sum(ex, axis=1, keepdims=True)
    den = jnp.where(den > 0.0, den, 1.0)
    alpha_p = ex / den

    bw = jnp.sum(alpha_p * amp, axis=1)
    bw4 = jnp.swapaxes(bw.reshape(G, _H, N), 1, 2).reshape(G * N, _H)
    bw128 = jnp.dot(bw4, hst_ref[...], precision=hi)

    alphaT = jnp.swapaxes(alpha_p, 1, 2)
    out_v = jnp.stack([
        jnp.concatenate([
            jnp.dot(alphaT[g, h * N:(h + 1) * N, :],
                    v[g, :, h * _C:(h + 1) * _C], precision=hi)
            for h in range(_H)], axis=1)
        for g in range(G)], axis=0)
    out_v = out_v.reshape(G * N, HC)

    out_ref[...] = jnp.maximum(out_v + bw128 * we + skip, 0.0)


@functools.partial(jax.jit, static_argnames=())
def kernel(node_obs, adj, W1, b1, W2, b2, Wq, bq, Wk, bk, Wv, bv, We,
           Wskip, bskip):
    M, N, D = node_obs.shape
    EH = W2.shape[0]
    HC = We.shape[1]
    G = _G
    f32 = jnp.float32

    hs = jnp.asarray(np.kron(np.eye(_H), np.ones((_C, 1))), f32)
    hst = jnp.asarray(np.kron(np.eye(_H), np.ones((1, _C))), f32)

    full = lambda shape: pl.BlockSpec(shape, lambda i: (0,) * len(shape))
    out = pl.pallas_call(
        _gnn_kernel,
        grid=(M // G,),
        in_specs=[
            pl.BlockSpec((G, N, D), lambda i: (i, 0, 0)),
            pl.BlockSpec((G, N, N), lambda i: (i, 0, 0)),
            full((D + 1, EH)),
            full((1, EH)),
            full((EH, EH)),
            full((1, EH)),
            full((EH, HC)),
            full((1, HC)),
            full((EH, HC)),
            full((1, HC)),
            full((EH, HC)),
            full((1, HC)),
            full((1, HC)),
            full((EH, HC)),
            full((1, HC)),
            full((HC, _H)),
            full((_H, HC)),
        ],
        out_specs=pl.BlockSpec((G * N, HC), lambda i: (i, 0)),
        out_shape=jax.ShapeDtypeStruct((M * N, HC), jnp.float32),
        scratch_shapes=[
            pltpu.VMEM((D, EH * N), f32),
            pltpu.VMEM((8, EH * N), f32),
            pltpu.VMEM((EH * N, EH * N), f32),
            pltpu.VMEM((EH, 4 * HC), f32),
            pltpu.VMEM((8, 4 * HC), f32),
        ],
    )(node_obs, adj, W1, b1.reshape(1, EH), W2, b2.reshape(1, EH),
      Wq, bq.reshape(1, HC), Wk, bk.reshape(1, HC), Wv, bv.reshape(1, HC),
      We, Wskip, bskip.reshape(1, HC), hs, hst)
    return out
